# baseline (speedup 1.0000x reference)
import functools

import jax
import jax.numpy as jnp
from jax import lax
from jax.experimental import pallas as pl
from jax.experimental.pallas import tpu as pltpu
from jax.experimental.pallas import tpu_sc as plsc

NC = 2
NS = 16
NW = NC * NS
GH = 64
GW = 64
BATCH = 256
VD = 64
PD = 64
ED = VD + PD
PLANE = GH * GW
TOTAL = BATCH * PLANE
PER_W = TOTAL // NW
CHUNK = 512
NCHUNK = PER_W // CHUNK
GSUB = 128
KSUB = CHUNK // GSUB
PLANES_PER_W = PER_W // PLANE


def _sc_body(idx_hbm, vtab_hbm, rowp_hbm, colp_hbm, out_hbm,
             idx_v, idx2_v, val0_v, val1_v, rowp_v, colp_v, pbuf_v,
             pos_sh, tab_sh, pos_hb, gsem, wsem0, wsem1, psem, isem0, isem1):
    cid = lax.axis_index("c")
    sid = lax.axis_index("s")
    wid = sid * NC + cid

    @pl.when(sid == 0)
    def _stage_table():
        pltpu.sync_copy(vtab_hbm, tab_sh)

    pltpu.sync_copy(rowp_hbm, rowp_v)
    pltpu.sync_copy(colp_hbm, colp_v)

    for hh in range(GH // NS):
        h = sid * (GH // NS) + hh

        def w_body(w, _):
            pbuf_v[w, pl.ds(0, 16)] = rowp_v[h, pl.ds(0, 16)]
            pbuf_v[w, pl.ds(16, 16)] = rowp_v[h, pl.ds(16, 16)]
            pbuf_v[w, pl.ds(32, 16)] = colp_v[w, pl.ds(0, 16)]
            pbuf_v[w, pl.ds(48, 16)] = colp_v[w, pl.ds(16, 16)]
            return 0

        lax.fori_loop(0, GW, w_body, 0)
        pltpu.sync_copy(pbuf_v, pos_sh.at[pl.ds(h * GW, GW)])
    plsc.subcore_barrier()

    @pl.when(sid == 0)
    def _stage_pos_hbm():
        pltpu.sync_copy(pos_sh, pos_hb)
    plsc.subcore_barrier()

    base0 = wid * PER_W

    def pos_dst(p):
        return out_hbm.at[pl.ds(base0 + p * PLANE, PLANE), pl.ds(VD, PD)]

    def pos_src(p):
        return pos_hb if p % 2 == 0 else pos_sh

    for p in range(PLANES_PER_W):
        pltpu.async_copy(pos_src(p), pos_dst(p), psem)

    def val_dst(cbase):
        return out_hbm.at[pl.ds(cbase, CHUNK), pl.ds(0, VD)]

    def idx_src(g):
        return idx_hbm.at[pl.ds(base0 // GSUB + g * KSUB, KSUB)]

    pltpu.async_copy(idx_src(0), idx_v, isem0)
    pltpu.async_copy(idx_src(1), idx2_v, isem1)

    def pair_body(t, _):
        for b, idx_b, isem, val_v, wsem in (
                (0, idx_v, isem0, val0_v, wsem0),
                (1, idx2_v, isem1, val1_v, wsem1)):
            g = 2 * t + b
            cbase = base0 + g * CHUNK
            pltpu.make_async_copy(idx_src(g), idx_b, isem).wait()

            @pl.when(t > 0)
            def _wait_prev_write():
                pltpu.make_async_copy(val_v, val_dst(cbase), wsem).wait()

            cps = [
                pltpu.async_copy(
                    tab_sh.at[idx_b.at[j]],
                    val_v.at[pl.ds(j * GSUB, GSUB)],
                    gsem)
                for j in range(KSUB)
            ]
            for cp in cps:
                cp.wait()

            @pl.when(t < NCHUNK // 2 - 1)
            def _prefetch_idx():
                pltpu.async_copy(idx_src(g + 2), idx_b, isem)

            pltpu.async_copy(val_v, val_dst(cbase), wsem)
        return 0

    lax.fori_loop(0, NCHUNK // 2, pair_body, 0)

    pltpu.make_async_copy(val0_v, val_dst(base0), wsem0).wait()
    pltpu.make_async_copy(val1_v, val_dst(base0), wsem1).wait()
    for p in range(PLANES_PER_W):
        pltpu.make_async_copy(pos_src(p), pos_dst(p), psem).wait()


_mesh = plsc.VectorSubcoreMesh(
    core_axis_name="c", subcore_axis_name="s", num_cores=NC, num_subcores=NS)

_sc_call = functools.partial(
    pl.kernel,
    out_type=jax.ShapeDtypeStruct((TOTAL, ED), jnp.float32),
    mesh=_mesh,
    scratch_types=[
        pltpu.VMEM((KSUB, GSUB), jnp.int32),
        pltpu.VMEM((KSUB, GSUB), jnp.int32),
        pltpu.VMEM((CHUNK, VD), jnp.float32),
        pltpu.VMEM((CHUNK, VD), jnp.float32),
        pltpu.VMEM((GH, 32), jnp.float32),
        pltpu.VMEM((GW, 32), jnp.float32),
        pltpu.VMEM((GW, PD), jnp.float32),
        pltpu.VMEM_SHARED((PLANE, PD), jnp.float32),
        pltpu.VMEM_SHARED((1024, VD), jnp.float32),
        pltpu.HBM((PLANE, PD), jnp.float32),
        pltpu.SemaphoreType.DMA,
        pltpu.SemaphoreType.DMA,
        pltpu.SemaphoreType.DMA,
        pltpu.SemaphoreType.DMA,
        pltpu.SemaphoreType.DMA,
        pltpu.SemaphoreType.DMA,
    ],
    compiler_params=pltpu.CompilerParams(use_tc_tiling_on_sc=False),
)(_sc_body)


@jax.jit
def kernel(grid, value_embed, row_pos_embed, col_pos_embed):
    idx = grid.astype(jnp.int32).reshape(TOTAL // GSUB, GSUB)
    out = _sc_call(idx, value_embed, row_pos_embed, col_pos_embed)
    return out.reshape(BATCH, GH, GW, ED)


# speedup vs baseline: 15.8329x; 15.8329x over previous
import functools

import jax
import jax.numpy as jnp
from jax import lax
from jax.experimental import pallas as pl
from jax.experimental.pallas import tpu as pltpu
from jax.experimental.pallas import tpu_sc as plsc

NC = 2
NS = 16
NW = NC * NS
GH = 64
GW = 64
BATCH = 256
VD = 64
PD = 64
ED = VD + PD
PLANE = GH * GW
TOTAL = BATCH * PLANE
PER_W = TOTAL // NW
CHUNK = 512
NCHUNK = PER_W // CHUNK
GSUB = 128
KSUB = CHUNK // GSUB
PLANES_PER_W = PER_W // PLANE


def _sc_body(idx_hbm, vtab_hbm, rowp_hbm, colp_hbm, out_hbm,
             idx_v, val0_v, val1_v, rowp_v, colp_v, pbuf_v,
             pos_sh, tab_sh, gsem0, gsem1, wsem0, wsem1, psem, isem):
    cid = lax.axis_index("c")
    sid = lax.axis_index("s")
    wid = sid * NC + cid
    base0 = wid * PER_W

    pltpu.async_copy(
        idx_hbm.at[pl.ds(base0 // GSUB, PER_W // GSUB)], idx_v, isem)

    @pl.when(sid == 0)
    def _stage_table():
        pltpu.sync_copy(vtab_hbm, tab_sh)

    pltpu.sync_copy(rowp_hbm, rowp_v)
    pltpu.sync_copy(colp_hbm, colp_v)

    for hh in range(GH // NS):
        h = sid * (GH // NS) + hh

        def w_body(w, _):
            pbuf_v[w, pl.ds(0, 16)] = rowp_v[h, pl.ds(0, 16)]
            pbuf_v[w, pl.ds(16, 16)] = rowp_v[h, pl.ds(16, 16)]
            pbuf_v[w, pl.ds(32, 16)] = colp_v[w, pl.ds(0, 16)]
            pbuf_v[w, pl.ds(48, 16)] = colp_v[w, pl.ds(16, 16)]
            return 0

        lax.fori_loop(0, GW, w_body, 0)
        pltpu.sync_copy(pbuf_v, pos_sh.at[pl.ds(h * GW, GW)])
    plsc.subcore_barrier()

    def pos_dst(p):
        return out_hbm.at[pl.ds(base0 + p * PLANE, PLANE), pl.ds(VD, PD)]

    for p in range(PLANES_PER_W):
        pltpu.async_copy(pos_sh, pos_dst(p), psem)

    def val_dst(g):
        return out_hbm.at[pl.ds(base0 + g * CHUNK, CHUNK), pl.ds(0, VD)]

    bufs = ((val0_v, gsem0, wsem0), (val1_v, gsem1, wsem1))

    def fire_gathers(g, val_v, gsem):
        for j in range(KSUB):
            pltpu.async_copy(
                tab_sh.at[idx_v.at[g * KSUB + j]],
                val_v.at[pl.ds(j * GSUB, GSUB)],
                gsem)

    def wait_gathers(val_v, gsem):
        pltpu.make_async_copy(
            tab_sh.at[pl.ds(0, CHUNK)], val_v, gsem).wait()

    def wait_write(g, val_v, wsem):
        pltpu.make_async_copy(val_v, val_dst(g), wsem).wait()

    pltpu.make_async_copy(
        idx_hbm.at[pl.ds(0, PER_W // GSUB)], idx_v, isem).wait()
    fire_gathers(0, val0_v, gsem0)

    def pair_body(t, _):
        for b in (0, 1):
            g = 2 * t + b
            val_b, gsem_b, wsem_b = bufs[b]
            val_n, gsem_n, wsem_n = bufs[1 - b]

            if b == 0:
                @pl.when(t > 0)
                def _wait_write_prev():
                    pltpu.make_async_copy(
                        val_n, val_dst(g), wsem_n).wait()
                fire_gathers(g + 1, val_n, gsem_n)
            else:
                @pl.when(t < NCHUNK // 2 - 1)
                def _fire_next():
                    wait_write(g, val_n, wsem_n)
                    fire_gathers(g + 1, val_n, gsem_n)

            wait_gathers(val_b, gsem_b)
            pltpu.async_copy(val_b, val_dst(g), wsem_b)
        return 0

    lax.fori_loop(0, NCHUNK // 2, pair_body, 0)

    wait_write(NCHUNK - 2, val0_v, wsem0)
    wait_write(NCHUNK - 1, val1_v, wsem1)
    for p in range(PLANES_PER_W):
        pltpu.make_async_copy(pos_sh, pos_dst(p), psem).wait()


_mesh = plsc.VectorSubcoreMesh(
    core_axis_name="c", subcore_axis_name="s", num_cores=NC, num_subcores=NS)

_sc_call = functools.partial(
    pl.kernel,
    out_type=jax.ShapeDtypeStruct((TOTAL, ED), jnp.float32),
    mesh=_mesh,
    scratch_types=[
        pltpu.VMEM((PER_W // GSUB, GSUB), jnp.int32),
        pltpu.VMEM((CHUNK, VD), jnp.float32),
        pltpu.VMEM((CHUNK, VD), jnp.float32),
        pltpu.VMEM((GH, 32), jnp.float32),
        pltpu.VMEM((GW, 32), jnp.float32),
        pltpu.VMEM((GW, PD), jnp.float32),
        pltpu.VMEM_SHARED((PLANE, PD), jnp.float32),
        pltpu.VMEM_SHARED((1024, VD), jnp.float32),
        pltpu.SemaphoreType.DMA,
        pltpu.SemaphoreType.DMA,
        pltpu.SemaphoreType.DMA,
        pltpu.SemaphoreType.DMA,
        pltpu.SemaphoreType.DMA,
        pltpu.SemaphoreType.DMA,
    ],
    compiler_params=pltpu.CompilerParams(use_tc_tiling_on_sc=False),
)(_sc_body)


@jax.jit
def kernel(grid, value_embed, row_pos_embed, col_pos_embed):
    idx = grid.astype(jnp.int32).reshape(TOTAL // GSUB, GSUB)
    out = _sc_call(idx, value_embed, row_pos_embed, col_pos_embed)
    return out.reshape(BATCH, GH, GW, ED)


# speedup vs baseline: 15.9145x; 1.0052x over previous
import functools

import jax
import jax.numpy as jnp
from jax import lax
from jax.experimental import pallas as pl
from jax.experimental.pallas import tpu as pltpu
from jax.experimental.pallas import tpu_sc as plsc

NC = 2
NS = 16
NW = NC * NS
GH = 64
GW = 64
BATCH = 256
VD = 64
PD = 64
ED = VD + PD
PLANE = GH * GW
TOTAL = BATCH * PLANE
PER_W = TOTAL // NW
CHUNK = 512
NCHUNK = PER_W // CHUNK
GSUB = 128
KSUB = CHUNK // GSUB
PLANES_PER_W = PER_W // PLANE


def _sc_body(idx_hbm, vtab_hbm, rowp_hbm, colp_hbm, out_hbm,
             idx_v, val0_v, val1_v, rowp_v, colp_v, pbuf_v,
             pos_sh, tab_sh, gsem0, gsem1, wsem0, wsem1, psem, isem):
    cid = lax.axis_index("c")
    sid = lax.axis_index("s")
    wid = sid * NC + cid
    base0 = wid * PER_W

    pltpu.async_copy(
        idx_hbm.at[pl.ds(base0 // GSUB, PER_W // GSUB)], idx_v, isem)

    @pl.when(sid == 0)
    def _stage_table():
        pltpu.sync_copy(vtab_hbm, tab_sh)

    pltpu.sync_copy(rowp_hbm, rowp_v)
    pltpu.sync_copy(colp_hbm, colp_v)

    for hh in range(GH // NS):
        h = sid * (GH // NS) + hh

        def w_body(w, _):
            pbuf_v[w, pl.ds(0, 16)] = rowp_v[h, pl.ds(0, 16)]
            pbuf_v[w, pl.ds(16, 16)] = rowp_v[h, pl.ds(16, 16)]
            pbuf_v[w, pl.ds(32, 16)] = colp_v[w, pl.ds(0, 16)]
            pbuf_v[w, pl.ds(48, 16)] = colp_v[w, pl.ds(16, 16)]
            return 0

        lax.fori_loop(0, GW, w_body, 0)
        pltpu.sync_copy(pbuf_v, pos_sh.at[pl.ds(h * GW, GW)])
    plsc.subcore_barrier()

    def pos_src(g):
        return pos_sh.at[pl.ds((g % (PLANE // CHUNK)) * CHUNK, CHUNK)]

    def pos_dst(g):
        return out_hbm.at[pl.ds(base0 + g * CHUNK, CHUNK), pl.ds(VD, PD)]

    def wait_pos(g):
        pltpu.make_async_copy(pos_src(g), pos_dst(g), psem).wait()

    def val_dst(g):
        return out_hbm.at[pl.ds(base0 + g * CHUNK, CHUNK), pl.ds(0, VD)]

    bufs = ((val0_v, gsem0, wsem0), (val1_v, gsem1, wsem1))

    def fire_gathers(g, val_v, gsem):
        for j in range(KSUB):
            pltpu.async_copy(
                tab_sh.at[idx_v.at[g * KSUB + j]],
                val_v.at[pl.ds(j * GSUB, GSUB)],
                gsem)

    def wait_gathers(val_v, gsem):
        pltpu.make_async_copy(
            tab_sh.at[pl.ds(0, CHUNK)], val_v, gsem).wait()

    def wait_write(g, val_v, wsem):
        pltpu.make_async_copy(val_v, val_dst(g), wsem).wait()

    pltpu.make_async_copy(
        idx_hbm.at[pl.ds(0, PER_W // GSUB)], idx_v, isem).wait()
    fire_gathers(0, val0_v, gsem0)

    def pair_body(t, _):
        for b in (0, 1):
            g = 2 * t + b
            val_b, gsem_b, wsem_b = bufs[b]
            val_n, gsem_n, wsem_n = bufs[1 - b]

            if b == 0:
                @pl.when(t > 0)
                def _wait_write_prev():
                    pltpu.make_async_copy(
                        val_n, val_dst(g), wsem_n).wait()
                    wait_pos(g)
                fire_gathers(g + 1, val_n, gsem_n)
            else:
                @pl.when(t < NCHUNK // 2 - 1)
                def _fire_next():
                    wait_write(g, val_n, wsem_n)
                    wait_pos(g)
                    fire_gathers(g + 1, val_n, gsem_n)

            wait_gathers(val_b, gsem_b)
            pltpu.async_copy(val_b, val_dst(g), wsem_b)
            pltpu.async_copy(pos_src(g), pos_dst(g), psem)
        return 0

    lax.fori_loop(0, NCHUNK // 2, pair_body, 0)

    wait_write(NCHUNK - 2, val0_v, wsem0)
    wait_write(NCHUNK - 1, val1_v, wsem1)
    wait_pos(NCHUNK - 2)
    wait_pos(NCHUNK - 1)


_mesh = plsc.VectorSubcoreMesh(
    core_axis_name="c", subcore_axis_name="s", num_cores=NC, num_subcores=NS)

_sc_call = functools.partial(
    pl.kernel,
    out_type=jax.ShapeDtypeStruct((TOTAL, ED), jnp.float32),
    mesh=_mesh,
    scratch_types=[
        pltpu.VMEM((PER_W // GSUB, GSUB), jnp.int32),
        pltpu.VMEM((CHUNK, VD), jnp.float32),
        pltpu.VMEM((CHUNK, VD), jnp.float32),
        pltpu.VMEM((GH, 32), jnp.float32),
        pltpu.VMEM((GW, 32), jnp.float32),
        pltpu.VMEM((GW, PD), jnp.float32),
        pltpu.VMEM_SHARED((PLANE, PD), jnp.float32),
        pltpu.VMEM_SHARED((1024, VD), jnp.float32),
        pltpu.SemaphoreType.DMA,
        pltpu.SemaphoreType.DMA,
        pltpu.SemaphoreType.DMA,
        pltpu.SemaphoreType.DMA,
        pltpu.SemaphoreType.DMA,
        pltpu.SemaphoreType.DMA,
    ],
    compiler_params=pltpu.CompilerParams(use_tc_tiling_on_sc=False),
)(_sc_body)


@jax.jit
def kernel(grid, value_embed, row_pos_embed, col_pos_embed):
    idx = grid.astype(jnp.int32).reshape(TOTAL // GSUB, GSUB)
    out = _sc_call(idx, value_embed, row_pos_embed, col_pos_embed)
    return out.reshape(BATCH, GH, GW, ED)
